# baseline (speedup 1.0000x reference)
import jax
import jax.numpy as jnp
from jax import lax
from jax.experimental import pallas as pl
from jax.experimental.pallas import tpu as pltpu
from jax.experimental.pallas import tpu_sc as plsc

B, T, C = 4, 8192, 1024
KK = 8
S = 16
G = T // S
NCB = 8
CB = C // NCB
NLG = CB // 16
NCG = 32
CGS = G // NCG
NEG_INF = float("-inf")


def _groupmax_body(x_ref, gm_ref, gmc_ref):
    acc = x_ref[0, pl.ds(0, G), :]
    for m in range(1, S):
        acc = jnp.maximum(acc, x_ref[0, pl.ds(m * G, G), :])
    gm_ref[0, 0] = acc
    cacc = acc[0:NCG, :]
    for j in range(1, CGS):
        cacc = jnp.maximum(cacc, acc[j * NCG:(j + 1) * NCG, :])
    gmc_ref[0, 0] = cacc


def _group_max(x):
    return pl.pallas_call(
        _groupmax_body,
        grid=(B, NCB),
        in_specs=[pl.BlockSpec((1, T, CB), lambda b, cb: (b, 0, cb))],
        out_specs=[
            pl.BlockSpec((1, 1, G, CB), lambda b, cb: (b, cb, 0, 0)),
            pl.BlockSpec((1, 1, NCG, CB), lambda b, cb: (b, cb, 0, 0)),
        ],
        out_shape=[
            jax.ShapeDtypeStruct((B, NCB, G, CB), jnp.float32),
            jax.ShapeDtypeStruct((B, NCB, NCG, CB), jnp.float32),
        ],
    )(x)


def _insert8(vals, x):
    out = []
    for i in range(KK):
        c = x > vals[i]
        out.append(jnp.where(c, x, vals[i]))
        x = jnp.where(c, vals[i], x)
    return out


def _insert8_kv(vals, idxs, x, g):
    out_v, out_i = [], []
    for i in range(KK):
        c = x > vals[i]
        out_v.append(jnp.where(c, x, vals[i]))
        out_i.append(jnp.where(c, g, idxs[i]))
        x = jnp.where(c, vals[i], x)
        g = jnp.where(c, idxs[i], g)
    return out_v, out_i


def _topk_sc_body(xflat_hbm, gm_hbm, gmc_hbm, out_hbm,
                  gm_v, gmc_v, candv, candg, out_v, *rest):
    idx_vs = rest[:NLG]
    gath_vs = rest[NLG:2 * NLG]
    sem = rest[2 * NLG]
    cid = lax.axis_index("c")
    sid = lax.axis_index("s")
    wid = sid * 2 + cid
    b = wid // NCB
    cb = lax.rem(wid, NCB)

    pltpu.sync_copy(gm_hbm.at[b, cb], gm_v)
    pltpu.sync_copy(gmc_hbm.at[b, cb], gmc_v)

    lanes = lax.iota(jnp.int32, 16)
    for lg in range(NLG):
        col = lg * 16

        def coarse_body(cg, carry):
            return tuple(_insert8(list(carry), gmc_v[cg, pl.ds(col, 16)]))
        top0 = lax.fori_loop(
            0, NCG, coarse_body,
            tuple(jnp.full((16,), NEG_INF, jnp.float32) for _ in range(KK)))
        tau0 = top0[KK - 1]

        def scan_body(r4, cnt):
            for u in range(4):
                r = r4 * 4 + u
                x = gm_v[r, pl.ds(col, 16)]
                msk = x >= tau0
                slot = cnt * 16 + lanes
                plsc.store_scatter(candv, [slot], x, mask=msk)
                plsc.store_scatter(candg, [slot],
                                   jnp.full((16,), r, jnp.int32), mask=msk)
                cnt = cnt + msk.astype(jnp.int32)
            return cnt
        cnt = lax.fori_loop(0, G // 4, scan_body, jnp.zeros((16,), jnp.int32))
        maxcnt = jnp.max(cnt)

        def ins_body(r, carry):
            vals = list(carry[:KK])
            idxs = list(carry[KK:])
            valid = r < cnt
            x = jnp.where(valid, candv[pl.ds(r * 16, 16)], NEG_INF)
            g = candg[pl.ds(r * 16, 16)]
            vals, idxs = _insert8_kv(vals, idxs, x, g)
            return tuple(vals) + tuple(idxs)
        init = (tuple(jnp.full((16,), NEG_INF, jnp.float32) for _ in range(KK))
                + tuple(jnp.zeros((16,), jnp.int32) for _ in range(KK)))
        res = lax.fori_loop(0, maxcnt, ins_body, init)
        gids = res[KK:]

        cbase = b * (T * C) + cb * CB + col + lanes
        for j in range(KK):
            base = gids[j] * C + cbase
            for m in range(S):
                idx_vs[lg][pl.ds((j * S + m) * 16, 16)] = base + m * (G * C)
        pltpu.async_copy(xflat_hbm.at[idx_vs[lg]], gath_vs[lg], sem)

    for lg in range(NLG):
        pltpu.make_async_copy(
            xflat_hbm.at[idx_vs[lg]], gath_vs[lg], sem).wait()

    for lg in range(NLG):
        col = lg * 16

        def fin_body(q4, carry):
            vals = list(carry)
            for u in range(4):
                x = gath_vs[lg][pl.ds((q4 * 4 + u) * 16, 16)]
                vals = _insert8(vals, x)
            return tuple(vals)
        top = lax.fori_loop(
            0, KK * S // 4, fin_body,
            tuple(jnp.full((16,), NEG_INF, jnp.float32) for _ in range(KK)))
        for k in range(KK):
            out_v[k, pl.ds(col, 16)] = top[k]

    pltpu.sync_copy(out_v, out_hbm.at[b, :, pl.ds(cb * CB, CB)])


def _topk_sc(xflat, gm, gmc):
    mesh = plsc.VectorSubcoreMesh(
        core_axis_name="c", subcore_axis_name="s", num_cores=2,
        num_subcores=16)
    f = pl.kernel(
        _topk_sc_body,
        out_type=jax.ShapeDtypeStruct((B, KK, C), jnp.float32),
        mesh=mesh,
        compiler_params=pltpu.CompilerParams(needs_layout_passes=False),
        scratch_types=[
            pltpu.VMEM((G, CB), jnp.float32),
            pltpu.VMEM((NCG, CB), jnp.float32),
            pltpu.VMEM((G * 16,), jnp.float32),
            pltpu.VMEM((G * 16,), jnp.int32),
            pltpu.VMEM((KK, CB), jnp.float32),
        ] + [pltpu.VMEM((KK * S * 16,), jnp.int32) for _ in range(NLG)]
          + [pltpu.VMEM((KK * S * 16,), jnp.float32) for _ in range(NLG)]
          + [pltpu.SemaphoreType.DMA],
    )
    return f(xflat, gm, gmc)


@jax.jit
def kernel(top_k):
    gm, gmc = _group_max(top_k)
    return _topk_sc(top_k.reshape(-1), gm, gmc)


# speedup vs baseline: 1.0051x; 1.0051x over previous
import jax
import jax.numpy as jnp
from jax import lax
from jax.experimental import pallas as pl
from jax.experimental.pallas import tpu as pltpu
from jax.experimental.pallas import tpu_sc as plsc

B, T, C = 4, 8192, 1024
KK = 8
S = 16
G = T // S
NCB = 8
CB = C // NCB
NLG = CB // 16
NCG = 32
CGS = G // NCG
NEG_INF = float("-inf")


def _groupmax_body(x_ref, gm_ref, gmc_ref):
    acc = x_ref[0, pl.ds(0, G), :]
    for m in range(1, S):
        acc = jnp.maximum(acc, x_ref[0, pl.ds(m * G, G), :])
    gm_ref[0, 0] = acc
    cacc = acc[0:NCG, :]
    for j in range(1, CGS):
        cacc = jnp.maximum(cacc, acc[j * NCG:(j + 1) * NCG, :])
    gmc_ref[0, 0] = cacc


def _group_max(x):
    return pl.pallas_call(
        _groupmax_body,
        grid=(B, NCB),
        in_specs=[pl.BlockSpec((1, T, CB), lambda b, cb: (b, 0, cb))],
        out_specs=[
            pl.BlockSpec((1, 1, G, CB), lambda b, cb: (b, cb, 0, 0)),
            pl.BlockSpec((1, 1, NCG, CB), lambda b, cb: (b, cb, 0, 0)),
        ],
        out_shape=[
            jax.ShapeDtypeStruct((B, NCB, G, CB), jnp.float32),
            jax.ShapeDtypeStruct((B, NCB, NCG, CB), jnp.float32),
        ],
    )(x)


def _insert8(vals, x):
    out = []
    for i in range(KK):
        c = x > vals[i]
        out.append(jnp.where(c, x, vals[i]))
        x = jnp.where(c, vals[i], x)
    return out


def _insert8_kv(vals, idxs, x, g):
    out_v, out_i = [], []
    for i in range(KK):
        c = x > vals[i]
        out_v.append(jnp.where(c, x, vals[i]))
        out_i.append(jnp.where(c, g, idxs[i]))
        x = jnp.where(c, vals[i], x)
        g = jnp.where(c, idxs[i], g)
    return out_v, out_i


def _topk_sc_body(xflat_hbm, gm_hbm, gmc_hbm, out_hbm,
                  gm_v, gmc_v, candv, candg, out_v, *rest):
    idx_vs = rest[:NLG]
    gath_vs = rest[NLG:2 * NLG]
    sem = rest[2 * NLG]
    cid = lax.axis_index("c")
    sid = lax.axis_index("s")
    wid = sid * 2 + cid
    b = wid // NCB
    cb = lax.rem(wid, NCB)

    pltpu.sync_copy(gm_hbm.at[b, cb], gm_v)
    pltpu.sync_copy(gmc_hbm.at[b, cb], gmc_v)

    lanes = lax.iota(jnp.int32, 16)
    for lg in range(NLG):
        col = lg * 16

        def coarse_body(cg, carry):
            return tuple(_insert8(list(carry), gmc_v[cg, pl.ds(col, 16)]))
        top0 = lax.fori_loop(
            0, NCG, coarse_body,
            tuple(jnp.full((16,), NEG_INF, jnp.float32) for _ in range(KK)))
        tau0 = top0[KK - 1]

        def scan_body(r, cnt):
            x = gm_v[r, pl.ds(col, 16)]
            msk = x >= tau0
            slot = cnt * 16 + lanes
            plsc.store_scatter(candv, [slot], x, mask=msk)
            plsc.store_scatter(candg, [slot],
                               jnp.full((16,), r, jnp.int32), mask=msk)
            return cnt + msk.astype(jnp.int32)
        cnt = lax.fori_loop(0, G, scan_body, jnp.zeros((16,), jnp.int32))
        maxcnt = jnp.max(cnt)

        def ins_body(r, carry):
            vals = list(carry[:KK])
            idxs = list(carry[KK:])
            valid = r < cnt
            x = jnp.where(valid, candv[pl.ds(r * 16, 16)], NEG_INF)
            g = candg[pl.ds(r * 16, 16)]
            vals, idxs = _insert8_kv(vals, idxs, x, g)
            return tuple(vals) + tuple(idxs)
        init = (tuple(jnp.full((16,), NEG_INF, jnp.float32) for _ in range(KK))
                + tuple(jnp.zeros((16,), jnp.int32) for _ in range(KK)))
        res = lax.fori_loop(0, maxcnt, ins_body, init)
        gids = res[KK:]

        cbase = b * (T * C) + cb * CB + col + lanes
        for j in range(KK):
            base = gids[j] * C + cbase
            for m in range(S):
                idx_vs[lg][pl.ds((j * S + m) * 16, 16)] = base + m * (G * C)
        pltpu.async_copy(xflat_hbm.at[idx_vs[lg]], gath_vs[lg], sem)

    for lg in range(NLG):
        pltpu.make_async_copy(
            xflat_hbm.at[idx_vs[lg]], gath_vs[lg], sem).wait()

    for lg in range(NLG):
        col = lg * 16

        def fin_body(q, carry):
            x = gath_vs[lg][pl.ds(q * 16, 16)]
            return tuple(_insert8(list(carry), x))
        top = lax.fori_loop(
            0, KK * S, fin_body,
            tuple(jnp.full((16,), NEG_INF, jnp.float32) for _ in range(KK)))
        for k in range(KK):
            out_v[k, pl.ds(col, 16)] = top[k]

    pltpu.sync_copy(out_v, out_hbm.at[b, :, pl.ds(cb * CB, CB)])


def _topk_sc(xflat, gm, gmc):
    mesh = plsc.VectorSubcoreMesh(
        core_axis_name="c", subcore_axis_name="s", num_cores=2,
        num_subcores=16)
    f = pl.kernel(
        _topk_sc_body,
        out_type=jax.ShapeDtypeStruct((B, KK, C), jnp.float32),
        mesh=mesh,
        compiler_params=pltpu.CompilerParams(needs_layout_passes=False),
        scratch_types=[
            pltpu.VMEM((G, CB), jnp.float32),
            pltpu.VMEM((NCG, CB), jnp.float32),
            pltpu.VMEM((G * 16,), jnp.float32),
            pltpu.VMEM((G * 16,), jnp.int32),
            pltpu.VMEM((KK, CB), jnp.float32),
        ] + [pltpu.VMEM((KK * S * 16,), jnp.int32) for _ in range(NLG)]
          + [pltpu.VMEM((KK * S * 16,), jnp.float32) for _ in range(NLG)]
          + [pltpu.SemaphoreType.DMA],
    )
    return f(xflat, gm, gmc)


@jax.jit
def kernel(top_k):
    gm, gmc = _group_max(top_k)
    return _topk_sc(top_k.reshape(-1), gm, gmc)
